# Initial kernel scaffold; baseline (speedup 1.0000x reference)
import jax
import jax.numpy as jnp
from jax.experimental import pallas as pl


def kernel(x, weight, bias, gamma, beta):
    raise NotImplementedError("write your pallas kernel here")


# speedup vs baseline: 5.1082x; 5.1082x over previous
import jax
import jax.numpy as jnp
from jax.experimental import pallas as pl
from jax.experimental.pallas import tpu as pltpu

_EPS = 1e-5
_NEG_SLOPE = 0.01

_H = 56
_C = 64
_F = 128
_KS = 3
_WP = 64
_HP = 64
_ROWS = _H * _WP


def _conv_rows(x_ref, w_ref):
    xf = x_ref[0]
    acc = jnp.zeros((_ROWS, _F), jnp.float32)
    for kh in range(_KS):
        for kw in range(_KS):
            off = kh * _WP + kw
            tap = kh * _KS + kw
            acc += jnp.dot(
                xf[off:off + _ROWS, :],
                w_ref[tap * _C:(tap + 1) * _C, :],
                preferred_element_type=jnp.float32,
            )
    return acc


def _stats_kernel(x_ref, w_ref, stats_ref):
    y = _conv_rows(x_ref, w_ref)
    yv = y.reshape(_H, _WP, _F)[:, :_H, :]
    s = jnp.sum(yv, axis=(0, 1)).reshape(1, _F)
    ss = jnp.sum(yv * yv, axis=(0, 1)).reshape(1, _F)
    stats_ref[...] = jnp.concatenate(
        [s, ss, jnp.zeros((6, _F), jnp.float32)], axis=0)


def _apply_kernel(x_ref, w_ref, scale_ref, shift_ref, o_ref):
    y = _conv_rows(x_ref, w_ref)
    o = y * scale_ref[...] + shift_ref[...]
    o = jnp.maximum(o, _NEG_SLOPE * o)
    o_ref[0] = o.reshape(_H, _WP, _F)[:, :_H, :]


@jax.jit
def _run(x, weight, gamma, beta):
    n = x.shape[0]
    m = n * _H * _H

    x_nhwc = jnp.transpose(x, (0, 2, 3, 1))
    x_pad = jnp.pad(x_nhwc, ((0, 0), (1, _HP - _H - 1), (1, _WP - _H - 1),
                             (0, 0)))
    x_flat = x_pad.reshape(n, _HP * _WP, _C)

    w_mat = jnp.transpose(weight, (2, 3, 1, 0)).reshape(_KS * _KS * _C, _F)

    grid = (n,)
    parallel = pltpu.CompilerParams(dimension_semantics=("parallel",))

    stats = pl.pallas_call(
        _stats_kernel,
        out_shape=jax.ShapeDtypeStruct((n * 8, _F), jnp.float32),
        grid=grid,
        in_specs=[
            pl.BlockSpec((1, _HP * _WP, _C), lambda i: (i, 0, 0)),
            pl.BlockSpec((_KS * _KS * _C, _F), lambda i: (0, 0)),
        ],
        out_specs=pl.BlockSpec((8, _F), lambda i: (i, 0)),
        compiler_params=parallel,
    )(x_flat, w_mat)

    stats = stats.reshape(n, 8, _F)
    inv_m = jnp.float32(1.0) / jnp.float32(m)
    mean = jnp.sum(stats[:, 0, :], axis=0) * inv_m
    var = jnp.maximum(jnp.sum(stats[:, 1, :], axis=0) * inv_m - mean * mean,
                      0.0)
    inv_std = jax.lax.rsqrt(var + _EPS)
    scale = (inv_std * gamma.astype(jnp.float32)).reshape(1, _F)
    shift = (beta.astype(jnp.float32) - mean * inv_std *
             gamma.astype(jnp.float32)).reshape(1, _F)

    out_nhwc = pl.pallas_call(
        _apply_kernel,
        out_shape=jax.ShapeDtypeStruct((n, _H, _H, _F), jnp.float32),
        grid=grid,
        in_specs=[
            pl.BlockSpec((1, _HP * _WP, _C), lambda i: (i, 0, 0)),
            pl.BlockSpec((_KS * _KS * _C, _F), lambda i: (0, 0)),
            pl.BlockSpec((1, _F), lambda i: (0, 0)),
            pl.BlockSpec((1, _F), lambda i: (0, 0)),
        ],
        out_specs=pl.BlockSpec((1, _H, _H, _F), lambda i: (i, 0, 0, 0)),
        compiler_params=parallel,
    )(x_flat, w_mat, scale, shift)

    return jnp.transpose(out_nhwc, (0, 3, 1, 2))


def kernel(x, weight, bias, gamma, beta):
    del bias
    return _run(x, weight, gamma, beta)
